# baseline (speedup 1.0000x reference)
import functools

import jax
import jax.numpy as jnp
from jax import lax
from jax.experimental import pallas as pl
from jax.experimental.pallas import tpu as pltpu
from jax.experimental.pallas import tpu_sc as plsc

_B = 32
_P = 24564
_PP = 24576
_NT = 16
_NCORE = 2
_NSUB = 16
_PW = _PP // _NSUB
_CH = _PW // 16
_BPC = _B // _NCORE
_VAR0 = 0.1
_VAR1 = 0.2
_SIGMA = 0.5
_LOG2 = 0.6931471805599453
_BIGI = 2 ** 30


def _gath(v, idx):
    return lax.gather(
        v, idx[:, None],
        dimension_numbers=lax.GatherDimensionNumbers(
            offset_dims=(), collapsed_slice_dims=(0,), start_index_map=(0,)),
        slice_sizes=(1,),
        mode=lax.GatherScatterMode.PROMISE_IN_BOUNDS)


def _perm(v, sh):
    return _gath(v, lax.iota(jnp.int32, 16) ^ sh)


def _bcast(v, t):
    return _gath(v, jnp.full((16,), t, jnp.int32))


def _allmax(v):
    for sh in (1, 2, 4, 8):
        v = jnp.maximum(v, _perm(v, sh))
    return v


def _allmin(v):
    for sh in (1, 2, 4, 8):
        v = jnp.minimum(v, _perm(v, sh))
    return v


def _log_atanh(x):
    x = jnp.maximum(x, 0.25)
    s = (x - 1.0) / (x + 1.0)
    s2 = s * s
    p = 1.0 + s2 * (1.0 / 3.0 + s2 * (1.0 / 5.0 + s2 * (1.0 / 7.0 + s2 * (1.0 / 9.0))))
    return 2.0 * s * p


def _sc_body(loc_h, pri_h, tru_h, out_loss_h, out_cnt_h,
             pri_v, pf_v, loc_v, truall_v, bestv_v, besti_v,
             resf_v, resi_v, allv_v, alli_v, of_v, oc_v,
             sh_max, sh_idx, dma_sem):
    cid = lax.axis_index("c")
    sid = lax.axis_index("s")
    base_p = sid * _PW
    iota = lax.iota(jnp.int32, 16)

    for c in range(4):
        pltpu.sync_copy(pri_h.at[pl.ds(c * _PP + base_p, _PW)], pri_v.at[c])
    pltpu.sync_copy(tru_h.at[pl.ds(cid * (_BPC * 64), _BPC * 64)], truall_v)

    def _pf(i, _):
        sl = pl.ds(i * 16, 16)
        cx = pri_v[0, sl]
        cy = pri_v[1, sl]
        w = pri_v[2, sl]
        h = pri_v[3, sl]
        pf_v[0, sl] = cx - w * 0.5
        pf_v[1, sl] = cy - h * 0.5
        pf_v[2, sl] = cx + w * 0.5
        pf_v[3, sl] = cy + h * 0.5
        pf_v[4, sl] = w * h
        return 0

    lax.fori_loop(0, _CH, _pf, 0)

    def _issue_loc(b, buf):
        for c in range(4):
            pltpu.async_copy(loc_h.at[b * 4 + c, pl.ds(base_p, _PW)],
                             loc_v.at[buf, c], dma_sem)

    def _drain_loc(buf):
        pltpu.make_async_copy(loc_h.at[pl.ds(0, 4), pl.ds(0, _PW)],
                              loc_v.at[buf], dma_sem).wait()

    _issue_loc(cid * _BPC * 1 + 0, 0)

    def _batch(bb, carry):
        lacc, cacc = carry
        b = cid * _BPC + bb
        buf = bb & 1

        resv = jnp.full((16,), -1.0, jnp.float32)
        resi = jnp.zeros((16,), jnp.int32)
        tx1v = truall_v[pl.ds(bb * 64, 16)]
        ty1v = truall_v[pl.ds(bb * 64 + 16, 16)]
        tx2v = truall_v[pl.ds(bb * 64 + 32, 16)]
        ty2v = truall_v[pl.ds(bb * 64 + 48, 16)]
        tav = (tx2v - tx1v) * (ty2v - ty1v)

        _GT = 8
        for g in range(_NT // _GT):

            def _chA(i, tc, g=g):
                sl = pl.ds(i * 16, 16)
                px1 = pf_v[0, sl]
                py1 = pf_v[1, sl]
                px2 = pf_v[2, sl]
                py2 = pf_v[3, sl]
                pa = pf_v[4, sl]
                pidx = base_p + i * 16 + iota
                if g == 0:
                    bv = jnp.full((16,), -1.0, jnp.float32)
                    bi = jnp.zeros((16,), jnp.int32)
                else:
                    bv = bestv_v[sl]
                    bi = besti_v[sl]
                out = []
                for k in range(_GT):
                    t = g * _GT + k
                    iw = jnp.minimum(px2, tx2v[t]) - jnp.maximum(px1, tx1v[t])
                    ih = jnp.minimum(py2, ty2v[t]) - jnp.maximum(py1, ty1v[t])
                    inter = jnp.maximum(iw, 0.0) * jnp.maximum(ih, 0.0)
                    ov = inter / (tav[t] + pa - inter)
                    m = ov > bv
                    bv = jnp.where(m, ov, bv)
                    bi = jnp.where(m, t, bi)
                    tbv, tbi = tc[2 * k], tc[2 * k + 1]
                    m2 = ov > tbv
                    out.append(jnp.where(m2, ov, tbv))
                    out.append(jnp.where(m2, pidx, tbi))
                bestv_v[sl] = bv
                besti_v[sl] = bi
                return tuple(out)

            init = []
            for k in range(_GT):
                init.append(jnp.full((16,), -1.0, jnp.float32))
                init.append(jnp.zeros((16,), jnp.int32))
            tc = lax.fori_loop(0, _CH, _chA, tuple(init))

            for k in range(_GT):
                t = g * _GT + k
                tbv, tbi = tc[2 * k], tc[2 * k + 1]
                mxv = _allmax(tbv)
                miv = _allmin(jnp.where(tbv == mxv, tbi, _BIGI))
                resv = jnp.where(iota == t, mxv, resv)
                resi = jnp.where(iota == t, miv, resi)

        resf_v[...] = resv
        resi_v[...] = resi
        pltpu.sync_copy(resf_v, sh_max.at[buf, sid])
        pltpu.sync_copy(resi_v, sh_idx.at[buf, sid])
        plsc.subcore_barrier()
        pltpu.sync_copy(sh_max.at[buf], allv_v)
        pltpu.sync_copy(sh_idx.at[buf], alli_v)

        gm = allv_v[0, :]
        for w in range(1, _NSUB):
            gm = jnp.maximum(gm, allv_v[w, :])
        gi = jnp.full((16,), _BIGI, jnp.int32)
        for w in range(_NSUB):
            m = allv_v[w, :] == gm
            gi = jnp.where(m, jnp.minimum(gi, alli_v[w, :]), gi)

        for t in range(_NT):
            lo = gi[t] - base_p
            inr = (lo >= 0) & (lo < _PW)

            @pl.when(inr)
            def _(lo=lo, t=t):
                c0 = (lo >> 4) << 4
                lane = iota == (lo & 15)
                sl = pl.ds(c0, 16)
                bestv_v[sl] = jnp.where(lane, 2.0, bestv_v[sl])
                besti_v[sl] = jnp.where(lane, t, besti_v[sl])

        _drain_loc(buf)

        @pl.when(bb < _BPC - 1)
        def _():
            _issue_loc(b + 1, 1 - buf)

        def _chB(i, c2):
            la, ca = c2
            sl = pl.ds(i * 16, 16)
            bv = bestv_v[sl]
            bi = besti_v[sl]
            pcx = pri_v[0, sl]
            pcy = pri_v[1, sl]
            pw = pri_v[2, sl]
            ph = pri_v[3, sl]
            dcx = pcx + loc_v[buf, 0, sl] * _VAR0 * pw
            dcy = pcy + loc_v[buf, 1, sl] * _VAR0 * ph
            dw = pw * jnp.exp(loc_v[buf, 2, sl] * _VAR1)
            dh = ph * jnp.exp(loc_v[buf, 3, sl] * _VAR1)
            dx1 = dcx - dw * 0.5
            dy1 = dcy - dh * 0.5
            dx2 = dcx + dw * 0.5
            dy2 = dcy + dh * 0.5
            gx1 = _gath(tx1v, bi)
            gy1 = _gath(ty1v, bi)
            gx2 = _gath(tx2v, bi)
            gy2 = _gath(ty2v, bi)
            ix1 = jnp.maximum(gx1, dx1)
            iy1 = jnp.maximum(gy1, dy1)
            ix2 = jnp.minimum(gx2, dx2)
            iy2 = jnp.minimum(gy2, dy2)
            inter = jnp.maximum(ix2 - ix1, 0.0) * jnp.maximum(iy2 - iy1, 0.0)
            g = (gx2 - gx1) * (gy2 - gy1)
            iog = inter / jnp.maximum(g, 1e-10)
            pos = bv >= 0.5
            valid = pos & (iog < 0.95)
            low = valid & (iog < _SIGMA)
            high = valid & (iog >= _SIGMA)
            tl = -_log_atanh(1.0 - iog + 1e-7)
            th = (iog - _SIGMA) * 2.0 + _LOG2
            la = la + jnp.where(low, tl, 0.0) + jnp.where(high, th, 0.0)
            ca = ca + jnp.where(pos, 1.0, 0.0)
            return (la, ca)

        return lax.fori_loop(0, _CH, _chB, (lacc, cacc))

    lacc, cacc = lax.fori_loop(
        0, _BPC, _batch,
        (jnp.zeros((16,), jnp.float32), jnp.zeros((16,), jnp.float32)))

    of_v[...] = lacc
    oc_v[...] = cacc
    wid = cid * _NSUB + sid
    pltpu.sync_copy(of_v, out_loss_h.at[wid])
    pltpu.sync_copy(oc_v, out_cnt_h.at[wid])


@jax.jit
def kernel(pred_loc, pred_score, priors, gt_data):
    del pred_score
    pad = _PP - _P

    loc_t = jnp.transpose(pred_loc, (0, 2, 1))
    loc_t = jnp.pad(loc_t, ((0, 0), (0, 0), (0, pad)))
    loc_f = loc_t.reshape(_B * 4, _PP)

    pri_t = jnp.transpose(priors, (1, 0))
    pad_box = jnp.array([[-100.0], [-100.0], [0.01], [0.01]], jnp.float32)
    pri_t = jnp.concatenate(
        [pri_t, jnp.broadcast_to(pad_box, (4, pad))], axis=1)
    pri_f = pri_t.reshape(-1)

    tru = jnp.transpose(gt_data[..., :4], (0, 2, 1)).reshape(-1)

    mesh = plsc.VectorSubcoreMesh(core_axis_name="c", subcore_axis_name="s",
                                  num_cores=_NCORE, num_subcores=_NSUB)
    run = pl.kernel(
        _sc_body,
        out_type=[jax.ShapeDtypeStruct((_NCORE * _NSUB, 16), jnp.float32),
                  jax.ShapeDtypeStruct((_NCORE * _NSUB, 16), jnp.float32)],
        mesh=mesh,
        scratch_types=[
            pltpu.VMEM((4, _PW), jnp.float32),
            pltpu.VMEM((5, _PW), jnp.float32),
            pltpu.VMEM((2, 4, _PW), jnp.float32),
            pltpu.VMEM((_BPC * 64,), jnp.float32),
            pltpu.VMEM((_PW,), jnp.float32),
            pltpu.VMEM((_PW,), jnp.int32),
            pltpu.VMEM((16,), jnp.float32),
            pltpu.VMEM((16,), jnp.int32),
            pltpu.VMEM((_NSUB, 16), jnp.float32),
            pltpu.VMEM((_NSUB, 16), jnp.int32),
            pltpu.VMEM((16,), jnp.float32),
            pltpu.VMEM((16,), jnp.float32),
            pltpu.VMEM_SHARED((2, _NSUB, 16), jnp.float32),
            pltpu.VMEM_SHARED((2, _NSUB, 16), jnp.int32),
            pltpu.SemaphoreType.DMA,
        ],
    )
    loss_p, cnt_p = run(loc_f, pri_f, tru)
    return jnp.sum(loss_p) / jnp.sum(cnt_p)


# speedup vs baseline: 1.2133x; 1.2133x over previous
import functools

import jax
import jax.numpy as jnp
from jax import lax
from jax.experimental import pallas as pl
from jax.experimental.pallas import tpu as pltpu
from jax.experimental.pallas import tpu_sc as plsc

_B = 32
_P = 24564
_PP = 24576
_NT = 16
_NCORE = 2
_NSUB = 16
_PW = _PP // _NSUB
_CH = _PW // 16
_BPC = _B // _NCORE
_VAR0 = 0.1
_VAR1 = 0.2
_SIGMA = 0.5
_LOG2 = 0.6931471805599453
_BIGI = 2 ** 30


def _gath(v, idx):
    return lax.gather(
        v, idx[:, None],
        dimension_numbers=lax.GatherDimensionNumbers(
            offset_dims=(), collapsed_slice_dims=(0,), start_index_map=(0,)),
        slice_sizes=(1,),
        mode=lax.GatherScatterMode.PROMISE_IN_BOUNDS)


def _perm(v, sh):
    return _gath(v, lax.iota(jnp.int32, 16) ^ sh)


def _bcast(v, t):
    return _gath(v, jnp.full((16,), t, jnp.int32))


def _allmax(v):
    for sh in (1, 2, 4, 8):
        v = jnp.maximum(v, _perm(v, sh))
    return v


def _allmin(v):
    for sh in (1, 2, 4, 8):
        v = jnp.minimum(v, _perm(v, sh))
    return v


def _log_atanh(x):
    x = jnp.maximum(x, 0.25)
    s = (x - 1.0) / (x + 1.0)
    s2 = s * s
    p = 1.0 + s2 * (1.0 / 3.0 + s2 * (1.0 / 5.0 + s2 * (1.0 / 7.0 + s2 * (1.0 / 9.0))))
    return 2.0 * s * p


def _sc_body(loc_h, pri_h, tru_h, out_loss_h, out_cnt_h,
             pri_v, pf_v, loc_v, truall_v, bestv_v, besti_v,
             resf_v, resi_v, allv_v, alli_v, of_v, oc_v,
             sh_max, sh_idx, dma_sem):
    cid = lax.axis_index("c")
    sid = lax.axis_index("s")
    base_p = sid * _PW
    iota = lax.iota(jnp.int32, 16)

    for c in range(4):
        pltpu.sync_copy(pri_h.at[pl.ds(c * _PP + base_p, _PW)], pri_v.at[c])
    pltpu.sync_copy(tru_h.at[pl.ds(cid * (_BPC * 64), _BPC * 64)], truall_v)

    def _pf(i, _):
        sl = pl.ds(i * 16, 16)
        cx = pri_v[0, sl]
        cy = pri_v[1, sl]
        w = pri_v[2, sl]
        h = pri_v[3, sl]
        pf_v[0, sl] = cx - w * 0.5
        pf_v[1, sl] = cy - h * 0.5
        pf_v[2, sl] = cx + w * 0.5
        pf_v[3, sl] = cy + h * 0.5
        pf_v[4, sl] = w * h
        return 0

    lax.fori_loop(0, _CH, _pf, 0)

    def _issue_loc(b, buf):
        for c in range(4):
            pltpu.async_copy(loc_h.at[b * 4 + c, pl.ds(base_p, _PW)],
                             loc_v.at[buf, c], dma_sem)

    def _drain_loc(buf):
        pltpu.make_async_copy(loc_h.at[pl.ds(0, 4), pl.ds(0, _PW)],
                              loc_v.at[buf], dma_sem).wait()

    _issue_loc(cid * _BPC * 1 + 0, 0)

    def _batch(bb, carry):
        lacc, cacc = carry
        b = cid * _BPC + bb
        buf = bb & 1

        resv = jnp.full((16,), -1.0, jnp.float32)
        resi = jnp.zeros((16,), jnp.int32)
        tx1v = truall_v[pl.ds(bb * 64, 16)]
        ty1v = truall_v[pl.ds(bb * 64 + 16, 16)]
        tx2v = truall_v[pl.ds(bb * 64 + 32, 16)]
        ty2v = truall_v[pl.ds(bb * 64 + 48, 16)]
        tav = (tx2v - tx1v) * (ty2v - ty1v)

        _GT = 4
        for g in range(_NT // _GT):

            def _chA(i, tc, g=g):
                sl = pl.ds(i * 16, 16)
                px1 = pf_v[0, sl]
                py1 = pf_v[1, sl]
                px2 = pf_v[2, sl]
                py2 = pf_v[3, sl]
                pa = pf_v[4, sl]
                pidx = base_p + i * 16 + iota
                if g == 0:
                    bv = jnp.full((16,), -1.0, jnp.float32)
                    bi = jnp.zeros((16,), jnp.int32)
                else:
                    bv = bestv_v[sl]
                    bi = besti_v[sl]
                out = []
                for k in range(_GT):
                    t = g * _GT + k
                    iw = jnp.minimum(px2, tx2v[t]) - jnp.maximum(px1, tx1v[t])
                    ih = jnp.minimum(py2, ty2v[t]) - jnp.maximum(py1, ty1v[t])
                    inter = jnp.maximum(iw, 0.0) * jnp.maximum(ih, 0.0)
                    ov = inter / (tav[t] + pa - inter)
                    m = ov > bv
                    bv = jnp.where(m, ov, bv)
                    bi = jnp.where(m, t, bi)
                    tbv, tbi = tc[2 * k], tc[2 * k + 1]
                    m2 = ov > tbv
                    out.append(jnp.where(m2, ov, tbv))
                    out.append(jnp.where(m2, pidx, tbi))
                bestv_v[sl] = bv
                besti_v[sl] = bi
                return tuple(out)

            init = []
            for k in range(_GT):
                init.append(jnp.full((16,), -1.0, jnp.float32))
                init.append(jnp.zeros((16,), jnp.int32))
            tc = lax.fori_loop(0, _CH, _chA, tuple(init), unroll=2)

            for k in range(_GT):
                t = g * _GT + k
                tbv, tbi = tc[2 * k], tc[2 * k + 1]
                mxv = _allmax(tbv)
                miv = _allmin(jnp.where(tbv == mxv, tbi, _BIGI))
                resv = jnp.where(iota == t, mxv, resv)
                resi = jnp.where(iota == t, miv, resi)

        resf_v[...] = resv
        resi_v[...] = resi
        pltpu.sync_copy(resf_v, sh_max.at[buf, sid])
        pltpu.sync_copy(resi_v, sh_idx.at[buf, sid])
        plsc.subcore_barrier()
        pltpu.sync_copy(sh_max.at[buf], allv_v)
        pltpu.sync_copy(sh_idx.at[buf], alli_v)

        gm = allv_v[0, :]
        for w in range(1, _NSUB):
            gm = jnp.maximum(gm, allv_v[w, :])
        gi = jnp.full((16,), _BIGI, jnp.int32)
        for w in range(_NSUB):
            m = allv_v[w, :] == gm
            gi = jnp.where(m, jnp.minimum(gi, alli_v[w, :]), gi)

        for t in range(_NT):
            lo = gi[t] - base_p
            inr = (lo >= 0) & (lo < _PW)

            @pl.when(inr)
            def _(lo=lo, t=t):
                c0 = (lo >> 4) << 4
                lane = iota == (lo & 15)
                sl = pl.ds(c0, 16)
                bestv_v[sl] = jnp.where(lane, 2.0, bestv_v[sl])
                besti_v[sl] = jnp.where(lane, t, besti_v[sl])

        _drain_loc(buf)

        @pl.when(bb < _BPC - 1)
        def _():
            _issue_loc(b + 1, 1 - buf)

        def _chB(i, c2):
            la, ca = c2
            sl = pl.ds(i * 16, 16)
            bv = bestv_v[sl]
            bi = besti_v[sl]
            pcx = pri_v[0, sl]
            pcy = pri_v[1, sl]
            pw = pri_v[2, sl]
            ph = pri_v[3, sl]
            dcx = pcx + loc_v[buf, 0, sl] * _VAR0 * pw
            dcy = pcy + loc_v[buf, 1, sl] * _VAR0 * ph
            dw = pw * jnp.exp(loc_v[buf, 2, sl] * _VAR1)
            dh = ph * jnp.exp(loc_v[buf, 3, sl] * _VAR1)
            dx1 = dcx - dw * 0.5
            dy1 = dcy - dh * 0.5
            dx2 = dcx + dw * 0.5
            dy2 = dcy + dh * 0.5
            gx1 = _gath(tx1v, bi)
            gy1 = _gath(ty1v, bi)
            gx2 = _gath(tx2v, bi)
            gy2 = _gath(ty2v, bi)
            ix1 = jnp.maximum(gx1, dx1)
            iy1 = jnp.maximum(gy1, dy1)
            ix2 = jnp.minimum(gx2, dx2)
            iy2 = jnp.minimum(gy2, dy2)
            inter = jnp.maximum(ix2 - ix1, 0.0) * jnp.maximum(iy2 - iy1, 0.0)
            g = (gx2 - gx1) * (gy2 - gy1)
            iog = inter / jnp.maximum(g, 1e-10)
            pos = bv >= 0.5
            valid = pos & (iog < 0.95)
            low = valid & (iog < _SIGMA)
            high = valid & (iog >= _SIGMA)
            tl = -_log_atanh(1.0 - iog + 1e-7)
            th = (iog - _SIGMA) * 2.0 + _LOG2
            la = la + jnp.where(low, tl, 0.0) + jnp.where(high, th, 0.0)
            ca = ca + jnp.where(pos, 1.0, 0.0)
            return (la, ca)

        return lax.fori_loop(0, _CH, _chB, (lacc, cacc))

    lacc, cacc = lax.fori_loop(
        0, _BPC, _batch,
        (jnp.zeros((16,), jnp.float32), jnp.zeros((16,), jnp.float32)))

    of_v[...] = lacc
    oc_v[...] = cacc
    wid = cid * _NSUB + sid
    pltpu.sync_copy(of_v, out_loss_h.at[wid])
    pltpu.sync_copy(oc_v, out_cnt_h.at[wid])


@jax.jit
def kernel(pred_loc, pred_score, priors, gt_data):
    del pred_score
    pad = _PP - _P

    loc_t = jnp.transpose(pred_loc, (0, 2, 1))
    loc_t = jnp.pad(loc_t, ((0, 0), (0, 0), (0, pad)))
    loc_f = loc_t.reshape(_B * 4, _PP)

    pri_t = jnp.transpose(priors, (1, 0))
    pad_box = jnp.array([[-100.0], [-100.0], [0.01], [0.01]], jnp.float32)
    pri_t = jnp.concatenate(
        [pri_t, jnp.broadcast_to(pad_box, (4, pad))], axis=1)
    pri_f = pri_t.reshape(-1)

    tru = jnp.transpose(gt_data[..., :4], (0, 2, 1)).reshape(-1)

    mesh = plsc.VectorSubcoreMesh(core_axis_name="c", subcore_axis_name="s",
                                  num_cores=_NCORE, num_subcores=_NSUB)
    run = pl.kernel(
        _sc_body,
        out_type=[jax.ShapeDtypeStruct((_NCORE * _NSUB, 16), jnp.float32),
                  jax.ShapeDtypeStruct((_NCORE * _NSUB, 16), jnp.float32)],
        mesh=mesh,
        scratch_types=[
            pltpu.VMEM((4, _PW), jnp.float32),
            pltpu.VMEM((5, _PW), jnp.float32),
            pltpu.VMEM((2, 4, _PW), jnp.float32),
            pltpu.VMEM((_BPC * 64,), jnp.float32),
            pltpu.VMEM((_PW,), jnp.float32),
            pltpu.VMEM((_PW,), jnp.int32),
            pltpu.VMEM((16,), jnp.float32),
            pltpu.VMEM((16,), jnp.int32),
            pltpu.VMEM((_NSUB, 16), jnp.float32),
            pltpu.VMEM((_NSUB, 16), jnp.int32),
            pltpu.VMEM((16,), jnp.float32),
            pltpu.VMEM((16,), jnp.float32),
            pltpu.VMEM_SHARED((2, _NSUB, 16), jnp.float32),
            pltpu.VMEM_SHARED((2, _NSUB, 16), jnp.int32),
            pltpu.SemaphoreType.DMA,
        ],
    )
    loss_p, cnt_p = run(loc_f, pri_f, tru)
    return jnp.sum(loss_p) / jnp.sum(cnt_p)


# speedup vs baseline: 1.2775x; 1.0530x over previous
import functools

import jax
import jax.numpy as jnp
from jax import lax
from jax.experimental import pallas as pl
from jax.experimental.pallas import tpu as pltpu
from jax.experimental.pallas import tpu_sc as plsc

_B = 32
_P = 24564
_PP = 24576
_NT = 16
_NCORE = 2
_NSUB = 16
_PW = _PP // _NSUB
_CH = _PW // 16
_BPC = _B // _NCORE
_VAR0 = 0.1
_VAR1 = 0.2
_SIGMA = 0.5
_LOG2 = 0.6931471805599453
_BIGI = 2 ** 30


def _gath(v, idx):
    return lax.gather(
        v, idx[:, None],
        dimension_numbers=lax.GatherDimensionNumbers(
            offset_dims=(), collapsed_slice_dims=(0,), start_index_map=(0,)),
        slice_sizes=(1,),
        mode=lax.GatherScatterMode.PROMISE_IN_BOUNDS)


def _perm(v, sh):
    return _gath(v, lax.iota(jnp.int32, 16) ^ sh)


def _bcast(v, t):
    return _gath(v, jnp.full((16,), t, jnp.int32))


def _allmax(v):
    for sh in (1, 2, 4, 8):
        v = jnp.maximum(v, _perm(v, sh))
    return v


def _allmin(v):
    for sh in (1, 2, 4, 8):
        v = jnp.minimum(v, _perm(v, sh))
    return v


def _log_atanh(x):
    x = jnp.maximum(x, 0.25)
    s = (x - 1.0) / (x + 1.0)
    s2 = s * s
    p = 1.0 + s2 * (1.0 / 3.0 + s2 * (1.0 / 5.0 + s2 * (1.0 / 7.0 + s2 * (1.0 / 9.0))))
    return 2.0 * s * p


def _sc_body(loc_h, pri_h, tru_h, out_loss_h, out_cnt_h,
             pri_v, pf_v, loc_v, truall_v, bestv_v, besti_v,
             resf_v, resi_v, allv_v, alli_v, of_v, oc_v,
             sh_max, sh_idx, dma_sem):
    cid = lax.axis_index("c")
    sid = lax.axis_index("s")
    base_p = sid * _PW
    iota = lax.iota(jnp.int32, 16)

    for c in range(4):
        pltpu.sync_copy(pri_h.at[pl.ds(c * _PP + base_p, _PW)], pri_v.at[c])
    pltpu.sync_copy(tru_h.at[pl.ds(cid * (_BPC * 64), _BPC * 64)], truall_v)

    def _pf(i, _):
        sl = pl.ds(i * 16, 16)
        cx = pri_v[0, sl]
        cy = pri_v[1, sl]
        w = pri_v[2, sl]
        h = pri_v[3, sl]
        pf_v[0, sl] = cx - w * 0.5
        pf_v[1, sl] = cy - h * 0.5
        pf_v[2, sl] = cx + w * 0.5
        pf_v[3, sl] = cy + h * 0.5
        pf_v[4, sl] = w * h
        return 0

    lax.fori_loop(0, _CH, _pf, 0)

    def _issue_loc(b, buf):
        for c in range(4):
            pltpu.async_copy(loc_h.at[b * 4 + c, pl.ds(base_p, _PW)],
                             loc_v.at[buf, c], dma_sem)

    def _drain_loc(buf):
        pltpu.make_async_copy(loc_h.at[pl.ds(0, 4), pl.ds(0, _PW)],
                              loc_v.at[buf], dma_sem).wait()

    _issue_loc(cid * _BPC * 1 + 0, 0)

    def _batch(bb, carry):
        lacc, cacc = carry
        b = cid * _BPC + bb
        buf = bb & 1

        resv = jnp.full((16,), -1.0, jnp.float32)
        resi = jnp.zeros((16,), jnp.int32)
        tx1v = truall_v[pl.ds(bb * 64, 16)]
        ty1v = truall_v[pl.ds(bb * 64 + 16, 16)]
        tx2v = truall_v[pl.ds(bb * 64 + 32, 16)]
        ty2v = truall_v[pl.ds(bb * 64 + 48, 16)]
        tav = (tx2v - tx1v) * (ty2v - ty1v)

        _GT = 4
        for g in range(_NT // _GT):

            def _chA(i, tc, g=g):
                sl = pl.ds(i * 16, 16)
                px1 = pf_v[0, sl]
                py1 = pf_v[1, sl]
                px2 = pf_v[2, sl]
                py2 = pf_v[3, sl]
                pa = pf_v[4, sl]
                pidx = base_p + i * 16 + iota
                if g == 0:
                    bv = jnp.full((16,), -1.0, jnp.float32)
                    bi = jnp.zeros((16,), jnp.int32)
                else:
                    bv = bestv_v[sl]
                    bi = besti_v[sl]
                out = []
                for k in range(_GT):
                    t = g * _GT + k
                    iw = jnp.minimum(px2, tx2v[t]) - jnp.maximum(px1, tx1v[t])
                    ih = jnp.minimum(py2, ty2v[t]) - jnp.maximum(py1, ty1v[t])
                    inter = jnp.maximum(iw, 0.0) * jnp.maximum(ih, 0.0)
                    ov = inter / (tav[t] + pa - inter)
                    m = ov > bv
                    bv = jnp.where(m, ov, bv)
                    bi = jnp.where(m, t, bi)
                    tbv, tbi = tc[2 * k], tc[2 * k + 1]
                    m2 = ov > tbv
                    out.append(jnp.where(m2, ov, tbv))
                    out.append(jnp.where(m2, pidx, tbi))
                bestv_v[sl] = bv
                besti_v[sl] = bi
                return tuple(out)

            init = []
            for k in range(_GT):
                init.append(jnp.full((16,), -1.0, jnp.float32))
                init.append(jnp.zeros((16,), jnp.int32))
            tc = lax.fori_loop(0, _CH, _chA, tuple(init))

            for k in range(_GT):
                t = g * _GT + k
                tbv, tbi = tc[2 * k], tc[2 * k + 1]
                mxv = _allmax(tbv)
                miv = _allmin(jnp.where(tbv == mxv, tbi, _BIGI))
                resv = jnp.where(iota == t, mxv, resv)
                resi = jnp.where(iota == t, miv, resi)

        resf_v[...] = resv
        resi_v[...] = resi
        pltpu.sync_copy(resf_v, sh_max.at[buf, sid])
        pltpu.sync_copy(resi_v, sh_idx.at[buf, sid])
        plsc.subcore_barrier()
        pltpu.sync_copy(sh_max.at[buf], allv_v)
        pltpu.sync_copy(sh_idx.at[buf], alli_v)

        gm = allv_v[0, :]
        for w in range(1, _NSUB):
            gm = jnp.maximum(gm, allv_v[w, :])
        gi = jnp.full((16,), _BIGI, jnp.int32)
        for w in range(_NSUB):
            m = allv_v[w, :] == gm
            gi = jnp.where(m, jnp.minimum(gi, alli_v[w, :]), gi)

        for t in range(_NT):
            lo = gi[t] - base_p
            inr = (lo >= 0) & (lo < _PW)

            @pl.when(inr)
            def _(lo=lo, t=t):
                c0 = (lo >> 4) << 4
                lane = iota == (lo & 15)
                sl = pl.ds(c0, 16)
                bestv_v[sl] = jnp.where(lane, 2.0, bestv_v[sl])
                besti_v[sl] = jnp.where(lane, t, besti_v[sl])

        _drain_loc(buf)

        @pl.when(bb < _BPC - 1)
        def _():
            _issue_loc(b + 1, 1 - buf)

        def _chB(i, c2):
            la, ca = c2
            sl = pl.ds(i * 16, 16)
            bv = bestv_v[sl]
            bi = besti_v[sl]
            pcx = pri_v[0, sl]
            pcy = pri_v[1, sl]
            pw = pri_v[2, sl]
            ph = pri_v[3, sl]
            dcx = pcx + loc_v[buf, 0, sl] * _VAR0 * pw
            dcy = pcy + loc_v[buf, 1, sl] * _VAR0 * ph
            dw = pw * jnp.exp(loc_v[buf, 2, sl] * _VAR1)
            dh = ph * jnp.exp(loc_v[buf, 3, sl] * _VAR1)
            dx1 = dcx - dw * 0.5
            dy1 = dcy - dh * 0.5
            dx2 = dcx + dw * 0.5
            dy2 = dcy + dh * 0.5
            gx1 = _gath(tx1v, bi)
            gy1 = _gath(ty1v, bi)
            gx2 = _gath(tx2v, bi)
            gy2 = _gath(ty2v, bi)
            ix1 = jnp.maximum(gx1, dx1)
            iy1 = jnp.maximum(gy1, dy1)
            ix2 = jnp.minimum(gx2, dx2)
            iy2 = jnp.minimum(gy2, dy2)
            inter = jnp.maximum(ix2 - ix1, 0.0) * jnp.maximum(iy2 - iy1, 0.0)
            g = (gx2 - gx1) * (gy2 - gy1)
            iog = inter / jnp.maximum(g, 1e-10)
            pos = bv >= 0.5
            valid = pos & (iog < 0.95)
            low = valid & (iog < _SIGMA)
            high = valid & (iog >= _SIGMA)
            tl = -_log_atanh(1.0 - iog + 1e-7)
            th = (iog - _SIGMA) * 2.0 + _LOG2
            la = la + jnp.where(low, tl, 0.0) + jnp.where(high, th, 0.0)
            ca = ca + jnp.where(pos, 1.0, 0.0)
            return (la, ca)

        return lax.fori_loop(0, _CH, _chB, (lacc, cacc), unroll=2)

    lacc, cacc = lax.fori_loop(
        0, _BPC, _batch,
        (jnp.zeros((16,), jnp.float32), jnp.zeros((16,), jnp.float32)))

    of_v[...] = lacc
    oc_v[...] = cacc
    wid = cid * _NSUB + sid
    pltpu.sync_copy(of_v, out_loss_h.at[wid])
    pltpu.sync_copy(oc_v, out_cnt_h.at[wid])


@jax.jit
def kernel(pred_loc, pred_score, priors, gt_data):
    del pred_score
    pad = _PP - _P

    loc_t = jnp.transpose(pred_loc, (0, 2, 1))
    loc_t = jnp.pad(loc_t, ((0, 0), (0, 0), (0, pad)))
    loc_f = loc_t.reshape(_B * 4, _PP)

    pri_t = jnp.transpose(priors, (1, 0))
    pad_box = jnp.array([[-100.0], [-100.0], [0.01], [0.01]], jnp.float32)
    pri_t = jnp.concatenate(
        [pri_t, jnp.broadcast_to(pad_box, (4, pad))], axis=1)
    pri_f = pri_t.reshape(-1)

    tru = jnp.transpose(gt_data[..., :4], (0, 2, 1)).reshape(-1)

    mesh = plsc.VectorSubcoreMesh(core_axis_name="c", subcore_axis_name="s",
                                  num_cores=_NCORE, num_subcores=_NSUB)
    run = pl.kernel(
        _sc_body,
        out_type=[jax.ShapeDtypeStruct((_NCORE * _NSUB, 16), jnp.float32),
                  jax.ShapeDtypeStruct((_NCORE * _NSUB, 16), jnp.float32)],
        mesh=mesh,
        scratch_types=[
            pltpu.VMEM((4, _PW), jnp.float32),
            pltpu.VMEM((5, _PW), jnp.float32),
            pltpu.VMEM((2, 4, _PW), jnp.float32),
            pltpu.VMEM((_BPC * 64,), jnp.float32),
            pltpu.VMEM((_PW,), jnp.float32),
            pltpu.VMEM((_PW,), jnp.int32),
            pltpu.VMEM((16,), jnp.float32),
            pltpu.VMEM((16,), jnp.int32),
            pltpu.VMEM((_NSUB, 16), jnp.float32),
            pltpu.VMEM((_NSUB, 16), jnp.int32),
            pltpu.VMEM((16,), jnp.float32),
            pltpu.VMEM((16,), jnp.float32),
            pltpu.VMEM_SHARED((2, _NSUB, 16), jnp.float32),
            pltpu.VMEM_SHARED((2, _NSUB, 16), jnp.int32),
            pltpu.SemaphoreType.DMA,
        ],
    )
    loss_p, cnt_p = run(loc_f, pri_f, tru)
    return jnp.sum(loss_p) / jnp.sum(cnt_p)


# speedup vs baseline: 1.2803x; 1.0022x over previous
import functools

import jax
import jax.numpy as jnp
from jax import lax
from jax.experimental import pallas as pl
from jax.experimental.pallas import tpu as pltpu
from jax.experimental.pallas import tpu_sc as plsc

_B = 32
_P = 24564
_PP = 24576
_NT = 16
_NCORE = 2
_NSUB = 16
_PW = _PP // _NSUB
_CH = _PW // 16
_BPC = _B // _NCORE
_VAR0 = 0.1
_VAR1 = 0.2
_SIGMA = 0.5
_LOG2 = 0.6931471805599453
_BIGI = 2 ** 30


def _gath(v, idx):
    return lax.gather(
        v, idx[:, None],
        dimension_numbers=lax.GatherDimensionNumbers(
            offset_dims=(), collapsed_slice_dims=(0,), start_index_map=(0,)),
        slice_sizes=(1,),
        mode=lax.GatherScatterMode.PROMISE_IN_BOUNDS)


def _perm(v, sh):
    return _gath(v, lax.iota(jnp.int32, 16) ^ sh)


def _bcast(v, t):
    return _gath(v, jnp.full((16,), t, jnp.int32))


def _allmax(v):
    for sh in (1, 2, 4, 8):
        v = jnp.maximum(v, _perm(v, sh))
    return v


def _allmin(v):
    for sh in (1, 2, 4, 8):
        v = jnp.minimum(v, _perm(v, sh))
    return v


def _log_atanh(x):
    x = jnp.maximum(x, 0.25)
    s = (x - 1.0) / (x + 1.0)
    s2 = s * s
    p = 1.0 + s2 * (1.0 / 3.0 + s2 * (1.0 / 5.0 + s2 * (1.0 / 7.0 + s2 * (1.0 / 9.0))))
    return 2.0 * s * p


def _sc_body(loc_h, pri_h, tru_h, out_loss_h, out_cnt_h,
             pri_v, pf_v, loc_v, truall_v, bestv_v, besti_v,
             resf_v, resi_v, allv_v, alli_v, of_v, oc_v,
             sh_max, sh_idx, dma_sem):
    cid = lax.axis_index("c")
    sid = lax.axis_index("s")
    base_p = sid * _PW
    iota = lax.iota(jnp.int32, 16)

    for c in range(4):
        pltpu.sync_copy(pri_h.at[pl.ds(c * _PP + base_p, _PW)], pri_v.at[c])
    pltpu.sync_copy(tru_h.at[pl.ds(cid * (_BPC * 64), _BPC * 64)], truall_v)

    def _pf(i, _):
        sl = pl.ds(i * 16, 16)
        cx = pri_v[0, sl]
        cy = pri_v[1, sl]
        w = pri_v[2, sl]
        h = pri_v[3, sl]
        pf_v[0, sl] = cx - w * 0.5
        pf_v[1, sl] = cy - h * 0.5
        pf_v[2, sl] = cx + w * 0.5
        pf_v[3, sl] = cy + h * 0.5
        pf_v[4, sl] = w * h
        return 0

    lax.fori_loop(0, _CH, _pf, 0)

    def _issue_loc(b, buf):
        for c in range(4):
            pltpu.async_copy(loc_h.at[b * 4 + c, pl.ds(base_p, _PW)],
                             loc_v.at[buf, c], dma_sem)

    def _drain_loc(buf):
        pltpu.make_async_copy(loc_h.at[pl.ds(0, 4), pl.ds(0, _PW)],
                              loc_v.at[buf], dma_sem).wait()

    _issue_loc(cid * _BPC * 1 + 0, 0)

    def _batch(bb, carry):
        lacc, cacc = carry
        b = cid * _BPC + bb
        buf = bb & 1

        resv = jnp.full((16,), -1.0, jnp.float32)
        resi = jnp.zeros((16,), jnp.int32)
        tx1v = truall_v[pl.ds(bb * 64, 16)]
        ty1v = truall_v[pl.ds(bb * 64 + 16, 16)]
        tx2v = truall_v[pl.ds(bb * 64 + 32, 16)]
        ty2v = truall_v[pl.ds(bb * 64 + 48, 16)]
        tav = (tx2v - tx1v) * (ty2v - ty1v)

        _GT = 4
        for g in range(_NT // _GT):

            def _chA(i, tc, g=g):
                sl = pl.ds(i * 16, 16)
                px1 = pf_v[0, sl]
                py1 = pf_v[1, sl]
                px2 = pf_v[2, sl]
                py2 = pf_v[3, sl]
                pa = pf_v[4, sl]
                pidx = base_p + i * 16 + iota
                if g == 0:
                    bv = jnp.full((16,), -1.0, jnp.float32)
                    bi = jnp.zeros((16,), jnp.int32)
                else:
                    bv = bestv_v[sl]
                    bi = besti_v[sl]
                out = []
                for k in range(_GT):
                    t = g * _GT + k
                    iw = jnp.minimum(px2, tx2v[t]) - jnp.maximum(px1, tx1v[t])
                    ih = jnp.minimum(py2, ty2v[t]) - jnp.maximum(py1, ty1v[t])
                    inter = jnp.maximum(iw, 0.0) * jnp.maximum(ih, 0.0)
                    ov = inter / (tav[t] + pa - inter)
                    m = ov > bv
                    bv = jnp.where(m, ov, bv)
                    bi = jnp.where(m, t, bi)
                    tbv, tbi = tc[2 * k], tc[2 * k + 1]
                    m2 = ov > tbv
                    out.append(jnp.where(m2, ov, tbv))
                    out.append(jnp.where(m2, pidx, tbi))
                bestv_v[sl] = bv
                besti_v[sl] = bi
                return tuple(out)

            init = []
            for k in range(_GT):
                init.append(jnp.full((16,), -1.0, jnp.float32))
                init.append(jnp.zeros((16,), jnp.int32))
            tc = lax.fori_loop(0, _CH, _chA, tuple(init))

            for k in range(_GT):
                t = g * _GT + k
                tbv, tbi = tc[2 * k], tc[2 * k + 1]
                mxv = _allmax(tbv)
                miv = _allmin(jnp.where(tbv == mxv, tbi, _BIGI))
                resv = jnp.where(iota == t, mxv, resv)
                resi = jnp.where(iota == t, miv, resi)

        resf_v[...] = resv
        resi_v[...] = resi
        pltpu.sync_copy(resf_v, sh_max.at[buf, sid])
        pltpu.sync_copy(resi_v, sh_idx.at[buf, sid])
        plsc.subcore_barrier()
        pltpu.sync_copy(sh_max.at[buf], allv_v)
        pltpu.sync_copy(sh_idx.at[buf], alli_v)

        gm = allv_v[0, :]
        for w in range(1, _NSUB):
            gm = jnp.maximum(gm, allv_v[w, :])
        gi = jnp.full((16,), _BIGI, jnp.int32)
        for w in range(_NSUB):
            m = allv_v[w, :] == gm
            gi = jnp.where(m, jnp.minimum(gi, alli_v[w, :]), gi)

        for t in range(_NT):
            lo = gi[t] - base_p
            inr = (lo >= 0) & (lo < _PW)

            @pl.when(inr)
            def _(lo=lo, t=t):
                c0 = (lo >> 4) << 4
                lane = iota == (lo & 15)
                sl = pl.ds(c0, 16)
                bestv_v[sl] = jnp.where(lane, 2.0, bestv_v[sl])
                besti_v[sl] = jnp.where(lane, t, besti_v[sl])

        _drain_loc(buf)

        @pl.when(bb < _BPC - 1)
        def _():
            _issue_loc(b + 1, 1 - buf)

        def _chB(i, c2):
            la, ca = c2
            sl = pl.ds(i * 16, 16)
            bv = bestv_v[sl]
            bi = besti_v[sl]
            pcx = pri_v[0, sl]
            pcy = pri_v[1, sl]
            pw = pri_v[2, sl]
            ph = pri_v[3, sl]
            dcx = pcx + loc_v[buf, 0, sl] * _VAR0 * pw
            dcy = pcy + loc_v[buf, 1, sl] * _VAR0 * ph
            dw = pw * jnp.exp(loc_v[buf, 2, sl] * _VAR1)
            dh = ph * jnp.exp(loc_v[buf, 3, sl] * _VAR1)
            dx1 = dcx - dw * 0.5
            dy1 = dcy - dh * 0.5
            dx2 = dcx + dw * 0.5
            dy2 = dcy + dh * 0.5
            gx1 = _gath(tx1v, bi)
            gy1 = _gath(ty1v, bi)
            gx2 = _gath(tx2v, bi)
            gy2 = _gath(ty2v, bi)
            ix1 = jnp.maximum(gx1, dx1)
            iy1 = jnp.maximum(gy1, dy1)
            ix2 = jnp.minimum(gx2, dx2)
            iy2 = jnp.minimum(gy2, dy2)
            inter = jnp.maximum(ix2 - ix1, 0.0) * jnp.maximum(iy2 - iy1, 0.0)
            g = (gx2 - gx1) * (gy2 - gy1)
            iog = inter / jnp.maximum(g, 1e-10)
            pos = bv >= 0.5
            valid = pos & (iog < 0.95)
            low = valid & (iog < _SIGMA)
            high = valid & (iog >= _SIGMA)
            tl = -_log_atanh(1.0 - iog + 1e-7)
            th = (iog - _SIGMA) * 2.0 + _LOG2
            la = la + jnp.where(low, tl, 0.0) + jnp.where(high, th, 0.0)
            ca = ca + jnp.where(pos, 1.0, 0.0)
            return (la, ca)

        return lax.fori_loop(0, _CH, _chB, (lacc, cacc))

    lacc, cacc = lax.fori_loop(
        0, _BPC, _batch,
        (jnp.zeros((16,), jnp.float32), jnp.zeros((16,), jnp.float32)))

    of_v[...] = lacc
    oc_v[...] = cacc
    wid = cid * _NSUB + sid
    pltpu.sync_copy(of_v, out_loss_h.at[wid])
    pltpu.sync_copy(oc_v, out_cnt_h.at[wid])


@jax.jit
def kernel(pred_loc, pred_score, priors, gt_data):
    del pred_score
    pad = _PP - _P

    loc_t = jnp.transpose(pred_loc, (0, 2, 1))
    loc_t = jnp.pad(loc_t, ((0, 0), (0, 0), (0, pad)))
    loc_f = loc_t.reshape(_B * 4, _PP)

    pri_t = jnp.transpose(priors, (1, 0))
    pad_box = jnp.array([[-100.0], [-100.0], [0.01], [0.01]], jnp.float32)
    pri_t = jnp.concatenate(
        [pri_t, jnp.broadcast_to(pad_box, (4, pad))], axis=1)
    pri_f = pri_t.reshape(-1)

    tru = jnp.transpose(gt_data[..., :4], (0, 2, 1)).reshape(-1)

    mesh = plsc.VectorSubcoreMesh(core_axis_name="c", subcore_axis_name="s",
                                  num_cores=_NCORE, num_subcores=_NSUB)
    run = pl.kernel(
        _sc_body,
        out_type=[jax.ShapeDtypeStruct((_NCORE * _NSUB, 16), jnp.float32),
                  jax.ShapeDtypeStruct((_NCORE * _NSUB, 16), jnp.float32)],
        mesh=mesh,
        scratch_types=[
            pltpu.VMEM((4, _PW), jnp.float32),
            pltpu.VMEM((5, _PW), jnp.float32),
            pltpu.VMEM((2, 4, _PW), jnp.float32),
            pltpu.VMEM((_BPC * 64,), jnp.float32),
            pltpu.VMEM((_PW,), jnp.float32),
            pltpu.VMEM((_PW,), jnp.int32),
            pltpu.VMEM((16,), jnp.float32),
            pltpu.VMEM((16,), jnp.int32),
            pltpu.VMEM((_NSUB, 16), jnp.float32),
            pltpu.VMEM((_NSUB, 16), jnp.int32),
            pltpu.VMEM((16,), jnp.float32),
            pltpu.VMEM((16,), jnp.float32),
            pltpu.VMEM_SHARED((2, _NSUB, 16), jnp.float32),
            pltpu.VMEM_SHARED((2, _NSUB, 16), jnp.int32),
            pltpu.SemaphoreType.DMA,
        ],
    )
    loss_p, cnt_p = run(loc_f, pri_f, tru)
    return jnp.sum(loss_p) / jnp.sum(cnt_p)
